# baseline (speedup 1.0000x reference)
import functools
import math

import jax
import jax.numpy as jnp
from jax import lax
from jax.experimental import pallas as pl
from jax.experimental.pallas import tpu as pltpu
from jax.experimental.pallas import tpu_sc as plsc

N = 10000
E = 320000
F = 128
NCLASS = 40
NLAYERS = 8
LAMDA = 0.5
ALPHA = 0.1

NC = 2
NS = 16
NTILES = NC * NS
CHUNK = 128
NCHT = 2560
NCH0 = NCHT // NS
HNCH = NCH0 // 2
EPAD = NCHT * CHUNK
NBUF = 2

ROWS_PT = 624
TAIL_BASE = NS * ROWS_PT
TAIL_ROWS = N - TAIL_BASE


def _spmm_body(h_hbm, pk_hbm, w_hbm, out_hbm,
               pkv, r0, r1, s0, s1, d0, d1, w0, w1,
               acc, gsem, wsem, ssem):
    rows = [r0, r1]
    sbuf = [s0, s1]
    dbuf = [d0, d1]
    wbuf = [w0, w1]
    c = lax.axis_index("c")
    s = lax.axis_index("s")
    rbase = s * ROWS_PT

    @pl.when(c == 0)
    def _zero():
        @pl.loop(0, CHUNK)
        def _z(i):
            for k in range(F // 16):
                r0[i, pl.ds(16 * k, 16)] = jnp.zeros((16,), jnp.float32)

        nfull = ROWS_PT // CHUNK
        rem = ROWS_PT - nfull * CHUNK
        for q in range(nfull):
            pltpu.sync_copy(r0, acc.at[pl.ds(rbase + q * CHUNK, CHUNK)])
        if rem:
            pltpu.sync_copy(r0.at[pl.ds(0, rem)],
                            acc.at[pl.ds(rbase + nfull * CHUNK, rem)])

        @pl.when(s == NS - 1)
        def _ztail():
            pltpu.sync_copy(r0.at[pl.ds(0, TAIL_ROWS)],
                            acc.at[pl.ds(TAIL_BASE, TAIL_ROWS)])

    plsc.subcore_barrier()

    def unpack(j, b):
        @pl.loop(0, CHUNK // 16)
        def _u(g):
            v = pkv[pl.ds(j * CHUNK + g * 16, 16)]
            sbuf[b][pl.ds(g * 16, 16)] = v & 0xFFFF
            dbuf[b][pl.ds(g * 16, 16)] = v >> 16

    def scatter_start(b):
        pltpu.async_copy(rows[b], acc.at[dbuf[b]], ssem.at[b], add=True)

    def scatter_wait(b):
        pltpu.make_async_copy(rows[b], acc.at[dbuf[b]], ssem.at[b]).wait()

    def run_edges(base_chunk, nch):
        pltpu.sync_copy(pk_hbm.at[pl.ds(base_chunk * CHUNK, nch * CHUNK)],
                        pkv.at[pl.ds(0, nch * CHUNK)])

        def issue(j, b):
            pltpu.async_copy(
                w_hbm.at[pl.ds((base_chunk + j) * CHUNK, CHUNK)],
                wbuf[b], wsem.at[b])
            pltpu.async_copy(h_hbm.at[sbuf[b]], rows[b], gsem.at[b])

        def wait_in(j, b):
            pltpu.make_async_copy(
                w_hbm.at[pl.ds((base_chunk + j) * CHUNK, CHUNK)],
                wbuf[b], wsem.at[b]).wait()
            pltpu.make_async_copy(h_hbm.at[sbuf[b]], rows[b], gsem.at[b]).wait()

        unpack(0, 0)
        issue(0, 0)

        @pl.loop(0, nch, step=NBUF)
        def _chunk(j0):
            for b in range(NBUF):
                j = j0 + b
                bn = (b + 1) % NBUF
                wait_in(j, b)

                jn = j + 1

                @pl.when(jn < nch)
                def _issue_next():
                    @pl.when(j >= 1)
                    def _drain_prev():
                        scatter_wait(bn)
                    unpack(jn, bn)
                    issue(jn, bn)

                @pl.loop(0, CHUNK // 16, unroll=2)
                def _scale(g):
                    wvec = wbuf[b][pl.ds(g * 16, 16)]
                    for i in range(16):
                        e = g * 16 + i
                        we = wvec[i]
                        for k in range(F // 16):
                            rows[b][e, pl.ds(16 * k, 16)] = (
                                rows[b][e, pl.ds(16 * k, 16)] * we)

                scatter_start(b)

        for b in range(NBUF):
            scatter_wait(b)

    @pl.when(c == 0)
    def _sc0():
        run_edges(s * NCH0, HNCH)
        run_edges(s * NCH0 + HNCH, HNCH)

    plsc.subcore_barrier()

    @pl.when(c == 0)
    def _out():
        pltpu.sync_copy(acc.at[pl.ds(rbase, ROWS_PT)], out_hbm.at[pl.ds(rbase, ROWS_PT)])

        @pl.when(s == NS - 1)
        def _otail():
            pltpu.sync_copy(acc.at[pl.ds(TAIL_BASE, TAIL_ROWS)],
                            out_hbm.at[pl.ds(TAIL_BASE, TAIL_ROWS)])


_spmm = pl.kernel(
    _spmm_body,
    out_type=jax.ShapeDtypeStruct((N, F), jnp.float32),
    mesh=plsc.VectorSubcoreMesh(core_axis_name="c", subcore_axis_name="s"),
    scratch_types=[
        pltpu.VMEM((HNCH * CHUNK,), jnp.int32),
        pltpu.VMEM((CHUNK, F), jnp.float32),
        pltpu.VMEM((CHUNK, F), jnp.float32),
        pltpu.VMEM((CHUNK,), jnp.int32),
        pltpu.VMEM((CHUNK,), jnp.int32),
        pltpu.VMEM((CHUNK,), jnp.int32),
        pltpu.VMEM((CHUNK,), jnp.int32),
        pltpu.VMEM((CHUNK,), jnp.float32),
        pltpu.VMEM((CHUNK,), jnp.float32),
        pltpu.VMEM_SHARED((N, F), jnp.float32),
        pltpu.SemaphoreType.DMA((NBUF,)),
        pltpu.SemaphoreType.DMA((NBUF,)),
        pltpu.SemaphoreType.DMA((NBUF,)),
    ],
)


_BM = 2000


def _fc0_body(x_ref, w_ref, b_ref, o_ref):
    o_ref[...] = jnp.maximum(
        jnp.dot(x_ref[...], w_ref[...], preferred_element_type=jnp.float32)
        + b_ref[...], 0.0)


def _fc0(x, W, b):
    return pl.pallas_call(
        _fc0_body,
        grid=(N // _BM,),
        in_specs=[
            pl.BlockSpec((_BM, F), lambda i: (i, 0)),
            pl.BlockSpec((F, F), lambda i: (0, 0)),
            pl.BlockSpec((1, F), lambda i: (0, 0)),
        ],
        out_specs=pl.BlockSpec((_BM, F), lambda i: (i, 0)),
        out_shape=jax.ShapeDtypeStruct((N, F), jnp.float32),
    )(x, W, b)


def _combine_body(theta, p_ref, h0_ref, w_ref, o_ref):
    support = (1.0 - ALPHA) * p_ref[...] + ALPHA * h0_ref[...]
    o_ref[...] = jnp.maximum(
        theta * jnp.dot(support, w_ref[...], preferred_element_type=jnp.float32)
        + (1.0 - theta) * support, 0.0)


def _combine(theta, parts, h0, W):
    return pl.pallas_call(
        functools.partial(_combine_body, theta),
        grid=(N // _BM,),
        in_specs=[
            pl.BlockSpec((_BM, F), lambda i: (i, 0)),
            pl.BlockSpec((_BM, F), lambda i: (i, 0)),
            pl.BlockSpec((F, F), lambda i: (0, 0)),
        ],
        out_specs=pl.BlockSpec((_BM, F), lambda i: (i, 0)),
        out_shape=jax.ShapeDtypeStruct((N, F), jnp.float32),
    )(parts, h0, W)


def _final_body(h_ref, w_ref, b_ref, o_ref):
    logits = (jnp.dot(h_ref[...], w_ref[...], preferred_element_type=jnp.float32)
              + b_ref[...])
    m = jnp.max(logits, axis=1, keepdims=True)
    ls = logits - m
    o_ref[...] = ls - jnp.log(jnp.sum(jnp.exp(ls), axis=1, keepdims=True))


def _final(h, Wp, bp):
    return pl.pallas_call(
        _final_body,
        grid=(N // _BM,),
        in_specs=[
            pl.BlockSpec((_BM, F), lambda i: (i, 0)),
            pl.BlockSpec((F, F), lambda i: (0, 0)),
            pl.BlockSpec((1, F), lambda i: (0, 0)),
        ],
        out_specs=pl.BlockSpec((_BM, F), lambda i: (i, 0)),
        out_shape=jax.ShapeDtypeStruct((N, F), jnp.float32),
    )(h, Wp, bp)


def kernel(x, edge_index, edge_weight, convW, fc0_W, fc0_b, fc1_W, fc1_b):
    src = edge_index[0]
    dst = edge_index[1]
    pad = EPAD - E
    packed = jnp.concatenate(
        [src | (dst << 16), jnp.zeros((pad,), jnp.int32)])
    wp = jnp.concatenate([edge_weight, jnp.zeros((pad,), jnp.float32)])

    h = _fc0(x, fc0_W, fc0_b.reshape(1, F))
    h0 = h
    for i in range(NLAYERS):
        parts = _spmm(h, packed, wp)
        theta = math.log(LAMDA / (i + 1) + 1.0)
        h = _combine(theta, parts, h0, convW[i])

    Wp = jnp.zeros((F, F), jnp.float32).at[:, :NCLASS].set(fc1_W)
    bp = jnp.full((1, F), -1e30, jnp.float32).at[0, :NCLASS].set(fc1_b)
    out = _final(h, Wp, bp)
    return out[:, :NCLASS]


# speedup vs baseline: 1.6697x; 1.6697x over previous
import functools
import math

import jax
import jax.numpy as jnp
from jax import lax
from jax.experimental import pallas as pl
from jax.experimental.pallas import tpu as pltpu
from jax.experimental.pallas import tpu_sc as plsc

N = 10000
E = 320000
F = 128
NCLASS = 40
NLAYERS = 8
LAMDA = 0.5
ALPHA = 0.1

NC = 2
NS = 16
NTILES = NC * NS
CHUNK = 128
NCHT = 2560
NCH0 = 144
HNCH = NCH0 // 2
NCH1 = NCHT // NS - NCH0
EPAD = NCHT * CHUNK
CH0_TOTAL = NS * NCH0
NBUF = 2

ROWS_PT = 624
TAIL_BASE = NS * ROWS_PT
TAIL_ROWS = N - TAIL_BASE


def _spmm_body(h_hbm, pk_hbm, w_hbm, out_hbm,
               pkv, r0, r1, s0, s1, d0, d1, w0, w1,
               acc, gsem, wsem, ssem):
    rows = [r0, r1]
    sbuf = [s0, s1]
    dbuf = [d0, d1]
    wbuf = [w0, w1]
    c = lax.axis_index("c")
    s = lax.axis_index("s")

    @pl.loop(0, CHUNK)
    def _z(i):
        for k in range(F // 16):
            r0[i, pl.ds(16 * k, 16)] = jnp.zeros((16,), jnp.float32)

    rbase = s * ROWS_PT
    nfull = ROWS_PT // CHUNK
    rem = ROWS_PT - nfull * CHUNK
    for q in range(nfull):
        pltpu.sync_copy(r0, acc.at[pl.ds(rbase + q * CHUNK, CHUNK)])
    if rem:
        pltpu.sync_copy(r0.at[pl.ds(0, rem)], acc.at[pl.ds(rbase + nfull * CHUNK, rem)])

    @pl.when(s == NS - 1)
    def _ztail():
        pltpu.sync_copy(r0.at[pl.ds(0, TAIL_ROWS)], acc.at[pl.ds(TAIL_BASE, TAIL_ROWS)])

    plsc.subcore_barrier()

    def unpack(j, b):
        @pl.loop(0, CHUNK // 16)
        def _u(g):
            v = pkv[pl.ds(j * CHUNK + g * 16, 16)]
            sbuf[b][pl.ds(g * 16, 16)] = v & 0xFFFF
            dbuf[b][pl.ds(g * 16, 16)] = v >> 16

    def scatter_start(b):
        pltpu.async_copy(rows[b], acc.at[dbuf[b]], ssem.at[b], add=True)

    def scatter_wait(b):
        pltpu.make_async_copy(rows[b], acc.at[dbuf[b]], ssem.at[b]).wait()

    def run_edges(base_chunk, nch):
        pltpu.sync_copy(pk_hbm.at[pl.ds(base_chunk * CHUNK, nch * CHUNK)],
                        pkv.at[pl.ds(0, nch * CHUNK)])

        def issue(j, b):
            pltpu.async_copy(
                w_hbm.at[pl.ds((base_chunk + j) * CHUNK, CHUNK)],
                wbuf[b], wsem.at[b])
            pltpu.async_copy(h_hbm.at[sbuf[b]], rows[b], gsem.at[b])

        def wait_in(j, b):
            pltpu.make_async_copy(
                w_hbm.at[pl.ds((base_chunk + j) * CHUNK, CHUNK)],
                wbuf[b], wsem.at[b]).wait()
            pltpu.make_async_copy(h_hbm.at[sbuf[b]], rows[b], gsem.at[b]).wait()

        unpack(0, 0)
        issue(0, 0)

        @pl.loop(0, nch, step=NBUF)
        def _chunk(j0):
            for b in range(NBUF):
                j = j0 + b
                bn = (b + 1) % NBUF
                wait_in(j, b)

                jn = j + 1

                @pl.when(jn < nch)
                def _issue_next():
                    @pl.when(j >= 1)
                    def _drain_prev():
                        scatter_wait(bn)
                    unpack(jn, bn)
                    issue(jn, bn)

                @pl.loop(0, CHUNK // 16, unroll=2)
                def _scale(g):
                    wvec = wbuf[b][pl.ds(g * 16, 16)]
                    for i in range(16):
                        e = g * 16 + i
                        we = wvec[i]
                        for k in range(F // 16):
                            rows[b][e, pl.ds(16 * k, 16)] = (
                                rows[b][e, pl.ds(16 * k, 16)] * we)

                scatter_start(b)

        for b in range(NBUF):
            scatter_wait(b)

    @pl.when(c == 0)
    def _sc0():
        run_edges(s * NCH0, HNCH)
        run_edges(s * NCH0 + HNCH, HNCH)

    @pl.when(c == 1)
    def _sc1():
        run_edges(CH0_TOTAL + s * NCH1, NCH1)

    plsc.subcore_barrier()
    pltpu.sync_copy(acc.at[pl.ds(rbase, ROWS_PT)], out_hbm.at[c, pl.ds(rbase, ROWS_PT)])

    @pl.when(s == NS - 1)
    def _otail():
        pltpu.sync_copy(acc.at[pl.ds(TAIL_BASE, TAIL_ROWS)],
                        out_hbm.at[c, pl.ds(TAIL_BASE, TAIL_ROWS)])


_spmm = pl.kernel(
    _spmm_body,
    out_type=jax.ShapeDtypeStruct((NC, N, F), jnp.float32),
    mesh=plsc.VectorSubcoreMesh(core_axis_name="c", subcore_axis_name="s"),
    scratch_types=[
        pltpu.VMEM((HNCH * CHUNK,), jnp.int32),
        pltpu.VMEM((CHUNK, F), jnp.float32),
        pltpu.VMEM((CHUNK, F), jnp.float32),
        pltpu.VMEM((CHUNK,), jnp.int32),
        pltpu.VMEM((CHUNK,), jnp.int32),
        pltpu.VMEM((CHUNK,), jnp.int32),
        pltpu.VMEM((CHUNK,), jnp.int32),
        pltpu.VMEM((CHUNK,), jnp.float32),
        pltpu.VMEM((CHUNK,), jnp.float32),
        pltpu.VMEM_SHARED((N, F), jnp.float32),
        pltpu.SemaphoreType.DMA((NBUF,)),
        pltpu.SemaphoreType.DMA((NBUF,)),
        pltpu.SemaphoreType.DMA((NBUF,)),
    ],
)


_BM = 2000


def _fc0_body(x_ref, w_ref, b_ref, o_ref):
    o_ref[...] = jnp.maximum(
        jnp.dot(x_ref[...], w_ref[...], preferred_element_type=jnp.float32)
        + b_ref[...], 0.0)


def _fc0(x, W, b):
    return pl.pallas_call(
        _fc0_body,
        grid=(N // _BM,),
        in_specs=[
            pl.BlockSpec((_BM, F), lambda i: (i, 0)),
            pl.BlockSpec((F, F), lambda i: (0, 0)),
            pl.BlockSpec((1, F), lambda i: (0, 0)),
        ],
        out_specs=pl.BlockSpec((_BM, F), lambda i: (i, 0)),
        out_shape=jax.ShapeDtypeStruct((N, F), jnp.float32),
    )(x, W, b)


def _combine_body(theta, p_ref, h0_ref, w_ref, o_ref):
    support = (1.0 - ALPHA) * (p_ref[0] + p_ref[1]) + ALPHA * h0_ref[...]
    o_ref[...] = jnp.maximum(
        theta * jnp.dot(support, w_ref[...], preferred_element_type=jnp.float32)
        + (1.0 - theta) * support, 0.0)


def _combine(theta, parts, h0, W):
    return pl.pallas_call(
        functools.partial(_combine_body, theta),
        grid=(N // _BM,),
        in_specs=[
            pl.BlockSpec((NC, _BM, F), lambda i: (0, i, 0)),
            pl.BlockSpec((_BM, F), lambda i: (i, 0)),
            pl.BlockSpec((F, F), lambda i: (0, 0)),
        ],
        out_specs=pl.BlockSpec((_BM, F), lambda i: (i, 0)),
        out_shape=jax.ShapeDtypeStruct((N, F), jnp.float32),
    )(parts, h0, W)


def _final_body(h_ref, w_ref, b_ref, o_ref):
    logits = (jnp.dot(h_ref[...], w_ref[...], preferred_element_type=jnp.float32)
              + b_ref[...])
    m = jnp.max(logits, axis=1, keepdims=True)
    ls = logits - m
    o_ref[...] = ls - jnp.log(jnp.sum(jnp.exp(ls), axis=1, keepdims=True))


def _final(h, Wp, bp):
    return pl.pallas_call(
        _final_body,
        grid=(N // _BM,),
        in_specs=[
            pl.BlockSpec((_BM, F), lambda i: (i, 0)),
            pl.BlockSpec((F, F), lambda i: (0, 0)),
            pl.BlockSpec((1, F), lambda i: (0, 0)),
        ],
        out_specs=pl.BlockSpec((_BM, F), lambda i: (i, 0)),
        out_shape=jax.ShapeDtypeStruct((N, F), jnp.float32),
    )(h, Wp, bp)


def kernel(x, edge_index, edge_weight, convW, fc0_W, fc0_b, fc1_W, fc1_b):
    src = edge_index[0]
    dst = edge_index[1]
    pad = EPAD - E
    packed = jnp.concatenate(
        [src | (dst << 16), jnp.zeros((pad,), jnp.int32)])
    wp = jnp.concatenate([edge_weight, jnp.zeros((pad,), jnp.float32)])

    h = _fc0(x, fc0_W, fc0_b.reshape(1, F))
    h0 = h
    for i in range(NLAYERS):
        parts = _spmm(h, packed, wp)
        theta = math.log(LAMDA / (i + 1) + 1.0)
        h = _combine(theta, parts, h0, convW[i])

    Wp = jnp.zeros((F, F), jnp.float32).at[:, :NCLASS].set(fc1_W)
    bp = jnp.full((1, F), -1e30, jnp.float32).at[0, :NCLASS].set(fc1_b)
    out = _final(h, Wp, bp)
    return out[:, :NCLASS]
